# baseline (speedup 1.0000x reference)
import functools

import jax
import jax.numpy as jnp
from jax import lax
from jax.experimental import pallas as pl
from jax.experimental.pallas import tpu as pltpu
from jax.experimental.pallas import tpu_sc as plsc

NUM_SEGMENTS = 16
ROWS_PER_SEG = 2048
COLS = 1024

K_SC = 0
M_TC = NUM_SEGMENTS - K_SC


TC_RB = 256
TC_NRB = ROWS_PER_SEG // TC_RB


def _tc_body(x_ref, o_ref):
    i = pl.program_id(0)
    r = pl.program_id(1)

    part = jnp.sum(x_ref[...], axis=0, keepdims=True)

    @pl.when(r == 0)
    def _init():
        o_ref[pl.ds(i, 1), :] = part

    @pl.when(r > 0)
    def _accum():
        o_ref[pl.ds(i, 1), :] += part

    @pl.when(r == TC_NRB - 1)
    def _scale():
        o_ref[pl.ds(i, 1), :] *= jnp.float32(1.0 / ROWS_PER_SEG)


def _tc_segmean(inp):
    return pl.pallas_call(
        _tc_body,
        grid=(M_TC, TC_NRB),
        in_specs=[pl.BlockSpec((TC_RB, COLS), lambda i, r: (i * TC_NRB + r, 0))],
        out_specs=pl.BlockSpec((M_TC, COLS), lambda i, r: (0, 0)),
        out_shape=jax.ShapeDtypeStruct((M_TC, COLS), jnp.float32),
    )(inp)


NC = 2
NS = 16
NW = NC * NS
NV = COLS // 16
RC = 32


def _make_sc_segmean(k_sc, seg0):
    w_per_seg = NW // k_sc
    rows_per_w = ROWS_PER_SEG // w_per_seg
    nch = rows_per_w // RC
    segs_per_core = k_sc // NC

    @functools.partial(
        pl.kernel,
        out_type=jax.ShapeDtypeStruct((k_sc, COLS), jnp.float32),
        mesh=plsc.VectorSubcoreMesh(core_axis_name="c", subcore_axis_name="s"),
        scratch_types=[
            pltpu.VMEM((RC, COLS), jnp.float32),
            pltpu.VMEM((RC, COLS), jnp.float32),
            pltpu.VMEM((COLS,), jnp.float32),
            pltpu.VMEM((COLS,), jnp.float32),
            pltpu.VMEM_SHARED((NS, COLS), jnp.float32),
            pltpu.SemaphoreType.DMA,
            pltpu.SemaphoreType.DMA,
            pltpu.SemaphoreType.DMA,
        ],
    )
    def _sc_segmean(inp_hbm, out_hbm, buf0, buf1, acc, pbuf, shared, sem0,
                    sem1, sem2):
        c = lax.axis_index("c")
        s = lax.axis_index("s")
        seg_local = c * segs_per_core + s // w_per_seg
        sub = s % w_per_seg
        row0 = (seg0 + seg_local) * ROWS_PER_SEG + sub * rows_per_w

        bufs = (buf0, buf1)
        sems = (sem0, sem1)

        def start(k, b):
            pltpu.make_async_copy(
                inp_hbm.at[pl.ds(row0 + k * RC, RC), :],
                bufs[b],
                sems[b],
            ).start()

        def wait(b):
            pltpu.make_async_copy(
                inp_hbm.at[pl.ds(row0, RC), :],
                bufs[b],
                sems[b],
            ).wait()

        def accum(buf):
            @plsc.parallel_loop(0, NV, unroll=2)
            def _jbody(j):
                cc = j * 16
                v = acc[pl.ds(cc, 16)]
                parts = []
                for g in range(RC // 8):
                    t = buf[g * 8, pl.ds(cc, 16)]
                    for r in range(g * 8 + 1, g * 8 + 8):
                        t = t + buf[r, pl.ds(cc, 16)]
                    parts.append(t)
                while len(parts) > 1:
                    parts = [a + b for a, b in zip(parts[::2], parts[1::2])]
                acc[pl.ds(cc, 16)] = v + parts[0]

        start(0, 0)
        start(1, 1)
        zero = jnp.zeros((16,), jnp.float32)
        for j in range(NV):
            acc[pl.ds(j * 16, 16)] = zero

        def ring(i, carry):
            for b in range(2):
                k = i * 2 + b
                wait(b)
                accum(bufs[b])
                start(k + 2, b)
            return carry
        lax.fori_loop(0, (nch - 2) // 2, ring, 0)

        wait(0)
        accum(buf0)
        wait(1)
        accum(buf1)

        pltpu.sync_copy(acc, shared.at[s])
        plsc.subcore_barrier()

        @pl.when(sub == 0)
        def _combine():
            for t in range(1, w_per_seg):
                pltpu.sync_copy(shared.at[s + t], pbuf)
                @plsc.parallel_loop(0, NV, unroll=2)
                def _addp(j):
                    cc = j * 16
                    acc[pl.ds(cc, 16)] = acc[pl.ds(cc, 16)] + pbuf[pl.ds(cc, 16)]
            scale = jnp.float32(1.0 / ROWS_PER_SEG)
            @plsc.parallel_loop(0, NV, unroll=2)
            def _scale(j):
                cc = j * 16
                acc[pl.ds(cc, 16)] = acc[pl.ds(cc, 16)] * scale
            pltpu.make_async_copy(acc, out_hbm.at[seg_local], sem2).start()
            pltpu.make_async_copy(acc, out_hbm.at[seg_local], sem2).wait()

    return _sc_segmean


if K_SC:
    _sc_call = _make_sc_segmean(K_SC, M_TC)


def kernel(inp):
    outs = []
    if M_TC:
        outs.append(_tc_segmean(inp))
    if K_SC:
        outs.append(_sc_call(inp))
    if len(outs) == 1:
        return outs[0]
    return jnp.concatenate(outs, axis=0)


# speedup vs baseline: 1.2728x; 1.2728x over previous
import functools

import jax
import jax.numpy as jnp
from jax import lax
from jax.experimental import pallas as pl
from jax.experimental.pallas import tpu as pltpu
from jax.experimental.pallas import tpu_sc as plsc

NUM_SEGMENTS = 16
ROWS_PER_SEG = 2048
COLS = 1024

K_SC = 8
M_TC = NUM_SEGMENTS - K_SC


TC_RB = 256
TC_NRB = ROWS_PER_SEG // TC_RB


def _tc_body(x_ref, o_ref):
    i = pl.program_id(0)
    r = pl.program_id(1)

    part = jnp.sum(x_ref[...], axis=0, keepdims=True)

    @pl.when(r == 0)
    def _init():
        o_ref[pl.ds(i, 1), :] = part

    @pl.when(r > 0)
    def _accum():
        o_ref[pl.ds(i, 1), :] += part

    @pl.when(r == TC_NRB - 1)
    def _scale():
        o_ref[pl.ds(i, 1), :] *= jnp.float32(1.0 / ROWS_PER_SEG)


def _tc_segmean(inp):
    return pl.pallas_call(
        _tc_body,
        grid=(M_TC, TC_NRB),
        in_specs=[pl.BlockSpec((TC_RB, COLS), lambda i, r: (i * TC_NRB + r, 0))],
        out_specs=pl.BlockSpec((M_TC, COLS), lambda i, r: (0, 0)),
        out_shape=jax.ShapeDtypeStruct((M_TC, COLS), jnp.float32),
    )(inp)


NC = 2
NS = 16
NW = NC * NS
NV = COLS // 16
RC = 32


def _make_sc_segmean(k_sc, seg0):
    w_per_seg = NW // k_sc
    rows_per_w = ROWS_PER_SEG // w_per_seg
    nch = rows_per_w // RC
    segs_per_core = k_sc // NC

    @functools.partial(
        pl.kernel,
        out_type=jax.ShapeDtypeStruct((k_sc, COLS), jnp.float32),
        mesh=plsc.VectorSubcoreMesh(core_axis_name="c", subcore_axis_name="s"),
        scratch_types=[
            pltpu.VMEM((RC, COLS), jnp.float32),
            pltpu.VMEM((RC, COLS), jnp.float32),
            pltpu.VMEM((COLS,), jnp.float32),
            pltpu.VMEM((COLS,), jnp.float32),
            pltpu.VMEM_SHARED((NS, COLS), jnp.float32),
            pltpu.SemaphoreType.DMA,
            pltpu.SemaphoreType.DMA,
            pltpu.SemaphoreType.DMA,
        ],
    )
    def _sc_segmean(inp_hbm, out_hbm, buf0, buf1, acc, pbuf, shared, sem0,
                    sem1, sem2):
        c = lax.axis_index("c")
        s = lax.axis_index("s")
        seg_local = c * segs_per_core + s // w_per_seg
        sub = s % w_per_seg
        row0 = (seg0 + seg_local) * ROWS_PER_SEG + sub * rows_per_w

        bufs = (buf0, buf1)
        sems = (sem0, sem1)

        def start(k, b):
            pltpu.make_async_copy(
                inp_hbm.at[pl.ds(row0 + k * RC, RC), :],
                bufs[b],
                sems[b],
            ).start()

        def wait(b):
            pltpu.make_async_copy(
                inp_hbm.at[pl.ds(row0, RC), :],
                bufs[b],
                sems[b],
            ).wait()

        def accum(buf):
            @plsc.parallel_loop(0, NV, unroll=2)
            def _jbody(j):
                cc = j * 16
                v = acc[pl.ds(cc, 16)]
                parts = []
                for g in range(RC // 8):
                    t = buf[g * 8, pl.ds(cc, 16)]
                    for r in range(g * 8 + 1, g * 8 + 8):
                        t = t + buf[r, pl.ds(cc, 16)]
                    parts.append(t)
                while len(parts) > 1:
                    parts = [a + b for a, b in zip(parts[::2], parts[1::2])]
                acc[pl.ds(cc, 16)] = v + parts[0]

        start(0, 0)
        start(1, 1)
        zero = jnp.zeros((16,), jnp.float32)
        for j in range(NV):
            acc[pl.ds(j * 16, 16)] = zero

        def ring(i, carry):
            for b in range(2):
                k = i * 2 + b
                wait(b)
                accum(bufs[b])
                start(k + 2, b)
            return carry
        lax.fori_loop(0, (nch - 2) // 2, ring, 0)

        wait(0)
        accum(buf0)
        wait(1)
        accum(buf1)

        pltpu.sync_copy(acc, shared.at[s])
        plsc.subcore_barrier()

        @pl.when(sub == 0)
        def _combine():
            for t in range(1, w_per_seg):
                pltpu.sync_copy(shared.at[s + t], pbuf)
                @plsc.parallel_loop(0, NV, unroll=2)
                def _addp(j):
                    cc = j * 16
                    acc[pl.ds(cc, 16)] = acc[pl.ds(cc, 16)] + pbuf[pl.ds(cc, 16)]
            scale = jnp.float32(1.0 / ROWS_PER_SEG)
            @plsc.parallel_loop(0, NV, unroll=2)
            def _scale(j):
                cc = j * 16
                acc[pl.ds(cc, 16)] = acc[pl.ds(cc, 16)] * scale
            pltpu.make_async_copy(acc, out_hbm.at[seg_local], sem2).start()
            pltpu.make_async_copy(acc, out_hbm.at[seg_local], sem2).wait()

    return _sc_segmean


if K_SC:
    _sc_call = _make_sc_segmean(K_SC, M_TC)


def kernel(inp):
    outs = []
    if M_TC:
        outs.append(_tc_segmean(inp))
    if K_SC:
        outs.append(_sc_call(inp))
    if len(outs) == 1:
        return outs[0]
    return jnp.concatenate(outs, axis=0)


# speedup vs baseline: 1.4063x; 1.1049x over previous
import functools

import jax
import jax.numpy as jnp
from jax import lax
from jax.experimental import pallas as pl
from jax.experimental.pallas import tpu as pltpu
from jax.experimental.pallas import tpu_sc as plsc

NUM_SEGMENTS = 16
ROWS_PER_SEG = 2048
COLS = 1024

K_SC = 8
M_TC = NUM_SEGMENTS - K_SC


TC_RB = 512
TC_NRB = ROWS_PER_SEG // TC_RB


def _tc_body(x_ref, o_ref):
    i = pl.program_id(0)
    r = pl.program_id(1)

    part = jnp.sum(x_ref[...], axis=0, keepdims=True)

    @pl.when(r == 0)
    def _init():
        o_ref[pl.ds(i, 1), :] = part

    @pl.when(r > 0)
    def _accum():
        o_ref[pl.ds(i, 1), :] += part

    @pl.when(r == TC_NRB - 1)
    def _scale():
        o_ref[pl.ds(i, 1), :] *= jnp.float32(1.0 / ROWS_PER_SEG)


def _tc_segmean(inp):
    return pl.pallas_call(
        _tc_body,
        grid=(M_TC, TC_NRB),
        in_specs=[pl.BlockSpec((TC_RB, COLS), lambda i, r: (i * TC_NRB + r, 0))],
        out_specs=pl.BlockSpec((M_TC, COLS), lambda i, r: (0, 0)),
        out_shape=jax.ShapeDtypeStruct((M_TC, COLS), jnp.float32),
    )(inp)


NC = 2
NS = 16
NW = NC * NS
NV = COLS // 16
RC = 32


def _make_sc_segmean(k_sc, seg0):
    w_per_seg = NW // k_sc
    rows_per_w = ROWS_PER_SEG // w_per_seg
    nch = rows_per_w // RC
    segs_per_core = k_sc // NC

    @functools.partial(
        pl.kernel,
        out_type=jax.ShapeDtypeStruct((k_sc, COLS), jnp.float32),
        mesh=plsc.VectorSubcoreMesh(core_axis_name="c", subcore_axis_name="s"),
        scratch_types=[
            pltpu.VMEM((RC, COLS), jnp.float32),
            pltpu.VMEM((RC, COLS), jnp.float32),
            pltpu.VMEM((COLS,), jnp.float32),
            pltpu.VMEM((COLS,), jnp.float32),
            pltpu.VMEM_SHARED((NS, COLS), jnp.float32),
            pltpu.SemaphoreType.DMA,
            pltpu.SemaphoreType.DMA,
            pltpu.SemaphoreType.DMA,
        ],
    )
    def _sc_segmean(inp_hbm, out_hbm, buf0, buf1, acc, pbuf, shared, sem0,
                    sem1, sem2):
        c = lax.axis_index("c")
        s = lax.axis_index("s")
        seg_local = c * segs_per_core + s // w_per_seg
        sub = s % w_per_seg
        row0 = (seg0 + seg_local) * ROWS_PER_SEG + sub * rows_per_w

        bufs = (buf0, buf1)
        sems = (sem0, sem1)

        def start(k, b):
            pltpu.make_async_copy(
                inp_hbm.at[pl.ds(row0 + k * RC, RC), :],
                bufs[b],
                sems[b],
            ).start()

        def wait(b):
            pltpu.make_async_copy(
                inp_hbm.at[pl.ds(row0, RC), :],
                bufs[b],
                sems[b],
            ).wait()

        def accum(buf):
            @plsc.parallel_loop(0, NV, unroll=2)
            def _jbody(j):
                cc = j * 16
                v = acc[pl.ds(cc, 16)]
                parts = []
                for g in range(RC // 8):
                    t = buf[g * 8, pl.ds(cc, 16)]
                    for r in range(g * 8 + 1, g * 8 + 8):
                        t = t + buf[r, pl.ds(cc, 16)]
                    parts.append(t)
                while len(parts) > 1:
                    parts = [a + b for a, b in zip(parts[::2], parts[1::2])]
                acc[pl.ds(cc, 16)] = v + parts[0]

        start(0, 0)
        start(1, 1)
        zero = jnp.zeros((16,), jnp.float32)
        for j in range(NV):
            acc[pl.ds(j * 16, 16)] = zero

        def ring(i, carry):
            for b in range(2):
                k = i * 2 + b
                wait(b)
                accum(bufs[b])
                start(k + 2, b)
            return carry
        lax.fori_loop(0, (nch - 2) // 2, ring, 0)

        wait(0)
        accum(buf0)
        wait(1)
        accum(buf1)

        pltpu.sync_copy(acc, shared.at[s])
        plsc.subcore_barrier()

        @pl.when(sub == 0)
        def _combine():
            for t in range(1, w_per_seg):
                pltpu.sync_copy(shared.at[s + t], pbuf)
                @plsc.parallel_loop(0, NV, unroll=2)
                def _addp(j):
                    cc = j * 16
                    acc[pl.ds(cc, 16)] = acc[pl.ds(cc, 16)] + pbuf[pl.ds(cc, 16)]
            scale = jnp.float32(1.0 / ROWS_PER_SEG)
            @plsc.parallel_loop(0, NV, unroll=2)
            def _scale(j):
                cc = j * 16
                acc[pl.ds(cc, 16)] = acc[pl.ds(cc, 16)] * scale
            pltpu.make_async_copy(acc, out_hbm.at[seg_local], sem2).start()
            pltpu.make_async_copy(acc, out_hbm.at[seg_local], sem2).wait()

    return _sc_segmean


if K_SC:
    _sc_call = _make_sc_segmean(K_SC, M_TC)


def kernel(inp):
    outs = []
    if M_TC:
        outs.append(_tc_segmean(inp))
    if K_SC:
        outs.append(_sc_call(inp))
    if len(outs) == 1:
        return outs[0]
    return jnp.concatenate(outs, axis=0)


# speedup vs baseline: 1.4189x; 1.0090x over previous
import functools

import jax
import jax.numpy as jnp
from jax import lax
from jax.experimental import pallas as pl
from jax.experimental.pallas import tpu as pltpu
from jax.experimental.pallas import tpu_sc as plsc

NUM_SEGMENTS = 16
ROWS_PER_SEG = 2048
COLS = 1024

K_SC = 8
M_TC = NUM_SEGMENTS - K_SC


TC_RB = 1024
TC_NRB = ROWS_PER_SEG // TC_RB


def _tc_body(x_ref, o_ref):
    i = pl.program_id(0)
    r = pl.program_id(1)

    part = jnp.sum(x_ref[...], axis=0, keepdims=True)

    @pl.when(r == 0)
    def _init():
        o_ref[pl.ds(i, 1), :] = part

    @pl.when(r > 0)
    def _accum():
        o_ref[pl.ds(i, 1), :] += part

    @pl.when(r == TC_NRB - 1)
    def _scale():
        o_ref[pl.ds(i, 1), :] *= jnp.float32(1.0 / ROWS_PER_SEG)


def _tc_segmean(inp):
    return pl.pallas_call(
        _tc_body,
        grid=(M_TC, TC_NRB),
        in_specs=[pl.BlockSpec((TC_RB, COLS), lambda i, r: (i * TC_NRB + r, 0))],
        out_specs=pl.BlockSpec((M_TC, COLS), lambda i, r: (0, 0)),
        out_shape=jax.ShapeDtypeStruct((M_TC, COLS), jnp.float32),
    )(inp)


NC = 2
NS = 16
NW = NC * NS
NV = COLS // 16
RC = 32


def _make_sc_segmean(k_sc, seg0):
    w_per_seg = NW // k_sc
    rows_per_w = ROWS_PER_SEG // w_per_seg
    nch = rows_per_w // RC
    segs_per_core = k_sc // NC

    @functools.partial(
        pl.kernel,
        out_type=jax.ShapeDtypeStruct((k_sc, COLS), jnp.float32),
        mesh=plsc.VectorSubcoreMesh(core_axis_name="c", subcore_axis_name="s"),
        scratch_types=[
            pltpu.VMEM((RC, COLS), jnp.float32),
            pltpu.VMEM((RC, COLS), jnp.float32),
            pltpu.VMEM((COLS,), jnp.float32),
            pltpu.VMEM((COLS,), jnp.float32),
            pltpu.VMEM_SHARED((NS, COLS), jnp.float32),
            pltpu.SemaphoreType.DMA,
            pltpu.SemaphoreType.DMA,
            pltpu.SemaphoreType.DMA,
        ],
    )
    def _sc_segmean(inp_hbm, out_hbm, buf0, buf1, acc, pbuf, shared, sem0,
                    sem1, sem2):
        c = lax.axis_index("c")
        s = lax.axis_index("s")
        seg_local = c * segs_per_core + s // w_per_seg
        sub = s % w_per_seg
        row0 = (seg0 + seg_local) * ROWS_PER_SEG + sub * rows_per_w

        bufs = (buf0, buf1)
        sems = (sem0, sem1)

        def start(k, b):
            pltpu.make_async_copy(
                inp_hbm.at[pl.ds(row0 + k * RC, RC), :],
                bufs[b],
                sems[b],
            ).start()

        def wait(b):
            pltpu.make_async_copy(
                inp_hbm.at[pl.ds(row0, RC), :],
                bufs[b],
                sems[b],
            ).wait()

        def accum(buf):
            @plsc.parallel_loop(0, NV, unroll=2)
            def _jbody(j):
                cc = j * 16
                v = acc[pl.ds(cc, 16)]
                parts = []
                for g in range(RC // 8):
                    t = buf[g * 8, pl.ds(cc, 16)]
                    for r in range(g * 8 + 1, g * 8 + 8):
                        t = t + buf[r, pl.ds(cc, 16)]
                    parts.append(t)
                while len(parts) > 1:
                    parts = [a + b for a, b in zip(parts[::2], parts[1::2])]
                acc[pl.ds(cc, 16)] = v + parts[0]

        start(0, 0)
        start(1, 1)
        zero = jnp.zeros((16,), jnp.float32)
        for j in range(NV):
            acc[pl.ds(j * 16, 16)] = zero

        def ring(i, carry):
            for b in range(2):
                k = i * 2 + b
                wait(b)
                accum(bufs[b])
                start(k + 2, b)
            return carry
        lax.fori_loop(0, (nch - 2) // 2, ring, 0)

        wait(0)
        accum(buf0)
        wait(1)
        accum(buf1)

        pltpu.sync_copy(acc, shared.at[s])
        plsc.subcore_barrier()

        @pl.when(sub == 0)
        def _combine():
            for t in range(1, w_per_seg):
                pltpu.sync_copy(shared.at[s + t], pbuf)
                @plsc.parallel_loop(0, NV, unroll=2)
                def _addp(j):
                    cc = j * 16
                    acc[pl.ds(cc, 16)] = acc[pl.ds(cc, 16)] + pbuf[pl.ds(cc, 16)]
            scale = jnp.float32(1.0 / ROWS_PER_SEG)
            @plsc.parallel_loop(0, NV, unroll=2)
            def _scale(j):
                cc = j * 16
                acc[pl.ds(cc, 16)] = acc[pl.ds(cc, 16)] * scale
            pltpu.make_async_copy(acc, out_hbm.at[seg_local], sem2).start()
            pltpu.make_async_copy(acc, out_hbm.at[seg_local], sem2).wait()

    return _sc_segmean


if K_SC:
    _sc_call = _make_sc_segmean(K_SC, M_TC)


def kernel(inp):
    outs = []
    if M_TC:
        outs.append(_tc_segmean(inp))
    if K_SC:
        outs.append(_sc_call(inp))
    if len(outs) == 1:
        return outs[0]
    return jnp.concatenate(outs, axis=0)


# speedup vs baseline: 1.4243x; 1.0038x over previous
import functools

import jax
import jax.numpy as jnp
from jax import lax
from jax.experimental import pallas as pl
from jax.experimental.pallas import tpu as pltpu
from jax.experimental.pallas import tpu_sc as plsc

NUM_SEGMENTS = 16
ROWS_PER_SEG = 2048
COLS = 1024

K_SC = 8
M_TC = NUM_SEGMENTS - K_SC


TC_RB = 2048
TC_NRB = ROWS_PER_SEG // TC_RB


def _tc_body(x_ref, o_ref):
    i = pl.program_id(0)
    r = pl.program_id(1)

    part = jnp.sum(x_ref[...], axis=0, keepdims=True)

    @pl.when(r == 0)
    def _init():
        o_ref[pl.ds(i, 1), :] = part

    @pl.when(r > 0)
    def _accum():
        o_ref[pl.ds(i, 1), :] += part

    @pl.when(r == TC_NRB - 1)
    def _scale():
        o_ref[pl.ds(i, 1), :] *= jnp.float32(1.0 / ROWS_PER_SEG)


def _tc_segmean(inp):
    return pl.pallas_call(
        _tc_body,
        grid=(M_TC, TC_NRB),
        in_specs=[pl.BlockSpec((TC_RB, COLS), lambda i, r: (i * TC_NRB + r, 0))],
        out_specs=pl.BlockSpec((M_TC, COLS), lambda i, r: (0, 0)),
        out_shape=jax.ShapeDtypeStruct((M_TC, COLS), jnp.float32),
    )(inp)


NC = 2
NS = 16
NW = NC * NS
NV = COLS // 16
RC = 32


def _make_sc_segmean(k_sc, seg0):
    w_per_seg = NW // k_sc
    rows_per_w = ROWS_PER_SEG // w_per_seg
    nch = rows_per_w // RC
    segs_per_core = k_sc // NC

    @functools.partial(
        pl.kernel,
        out_type=jax.ShapeDtypeStruct((k_sc, COLS), jnp.float32),
        mesh=plsc.VectorSubcoreMesh(core_axis_name="c", subcore_axis_name="s"),
        scratch_types=[
            pltpu.VMEM((RC, COLS), jnp.float32),
            pltpu.VMEM((RC, COLS), jnp.float32),
            pltpu.VMEM((COLS,), jnp.float32),
            pltpu.VMEM((COLS,), jnp.float32),
            pltpu.VMEM_SHARED((NS, COLS), jnp.float32),
            pltpu.SemaphoreType.DMA,
            pltpu.SemaphoreType.DMA,
            pltpu.SemaphoreType.DMA,
        ],
    )
    def _sc_segmean(inp_hbm, out_hbm, buf0, buf1, acc, pbuf, shared, sem0,
                    sem1, sem2):
        c = lax.axis_index("c")
        s = lax.axis_index("s")
        seg_local = c * segs_per_core + s // w_per_seg
        sub = s % w_per_seg
        row0 = (seg0 + seg_local) * ROWS_PER_SEG + sub * rows_per_w

        bufs = (buf0, buf1)
        sems = (sem0, sem1)

        def start(k, b):
            pltpu.make_async_copy(
                inp_hbm.at[pl.ds(row0 + k * RC, RC), :],
                bufs[b],
                sems[b],
            ).start()

        def wait(b):
            pltpu.make_async_copy(
                inp_hbm.at[pl.ds(row0, RC), :],
                bufs[b],
                sems[b],
            ).wait()

        def accum(buf):
            @plsc.parallel_loop(0, NV, unroll=2)
            def _jbody(j):
                cc = j * 16
                v = acc[pl.ds(cc, 16)]
                parts = []
                for g in range(RC // 8):
                    t = buf[g * 8, pl.ds(cc, 16)]
                    for r in range(g * 8 + 1, g * 8 + 8):
                        t = t + buf[r, pl.ds(cc, 16)]
                    parts.append(t)
                while len(parts) > 1:
                    parts = [a + b for a, b in zip(parts[::2], parts[1::2])]
                acc[pl.ds(cc, 16)] = v + parts[0]

        start(0, 0)
        start(1, 1)
        zero = jnp.zeros((16,), jnp.float32)
        for j in range(NV):
            acc[pl.ds(j * 16, 16)] = zero

        def ring(i, carry):
            for b in range(2):
                k = i * 2 + b
                wait(b)
                accum(bufs[b])
                start(k + 2, b)
            return carry
        lax.fori_loop(0, (nch - 2) // 2, ring, 0)

        wait(0)
        accum(buf0)
        wait(1)
        accum(buf1)

        pltpu.sync_copy(acc, shared.at[s])
        plsc.subcore_barrier()

        @pl.when(sub == 0)
        def _combine():
            for t in range(1, w_per_seg):
                pltpu.sync_copy(shared.at[s + t], pbuf)
                @plsc.parallel_loop(0, NV, unroll=2)
                def _addp(j):
                    cc = j * 16
                    acc[pl.ds(cc, 16)] = acc[pl.ds(cc, 16)] + pbuf[pl.ds(cc, 16)]
            scale = jnp.float32(1.0 / ROWS_PER_SEG)
            @plsc.parallel_loop(0, NV, unroll=2)
            def _scale(j):
                cc = j * 16
                acc[pl.ds(cc, 16)] = acc[pl.ds(cc, 16)] * scale
            pltpu.make_async_copy(acc, out_hbm.at[seg_local], sem2).start()
            pltpu.make_async_copy(acc, out_hbm.at[seg_local], sem2).wait()

    return _sc_segmean


if K_SC:
    _sc_call = _make_sc_segmean(K_SC, M_TC)


def kernel(inp):
    outs = []
    if M_TC:
        outs.append(_tc_segmean(inp))
    if K_SC:
        outs.append(_sc_call(inp))
    if len(outs) == 1:
        return outs[0]
    return jnp.concatenate(outs, axis=0)


# speedup vs baseline: 1.4622x; 1.0266x over previous
import functools

import jax
import jax.numpy as jnp
from jax import lax
from jax.experimental import pallas as pl
from jax.experimental.pallas import tpu as pltpu
from jax.experimental.pallas import tpu_sc as plsc

NUM_SEGMENTS = 16
ROWS_PER_SEG = 2048
COLS = 1024

K_SC = 8
M_TC = NUM_SEGMENTS - K_SC


TC_RB = 2048
TC_NRB = ROWS_PER_SEG // TC_RB


def _tc_body(x_ref, o_ref):
    i = pl.program_id(0)
    r = pl.program_id(1)

    part = jnp.sum(x_ref[...], axis=0, keepdims=True)

    @pl.when(r == 0)
    def _init():
        o_ref[pl.ds(i, 1), :] = part

    @pl.when(r > 0)
    def _accum():
        o_ref[pl.ds(i, 1), :] += part

    @pl.when(r == TC_NRB - 1)
    def _scale():
        o_ref[pl.ds(i, 1), :] *= jnp.float32(1.0 / ROWS_PER_SEG)


def _tc_segmean(inp):
    return pl.pallas_call(
        _tc_body,
        grid=(M_TC, TC_NRB),
        in_specs=[pl.BlockSpec((TC_RB, COLS), lambda i, r: (i * TC_NRB + r, 0))],
        out_specs=pl.BlockSpec((M_TC, COLS), lambda i, r: (0, 0)),
        out_shape=jax.ShapeDtypeStruct((M_TC, COLS), jnp.float32),
    )(inp)


NC = 2
NS = 16
NW = NC * NS
NV = COLS // 16
RC = 16
NBUF = 4


def _make_sc_segmean(k_sc, seg0):
    w_per_seg = NW // k_sc
    rows_per_w = ROWS_PER_SEG // w_per_seg
    nch = rows_per_w // RC
    segs_per_core = k_sc // NC

    @functools.partial(
        pl.kernel,
        out_type=jax.ShapeDtypeStruct((k_sc, COLS), jnp.float32),
        mesh=plsc.VectorSubcoreMesh(core_axis_name="c", subcore_axis_name="s"),
        scratch_types=(
            [pltpu.VMEM((RC, COLS), jnp.float32) for _ in range(NBUF)]
            + [
                pltpu.VMEM((COLS,), jnp.float32),
                pltpu.VMEM((COLS,), jnp.float32),
                pltpu.VMEM_SHARED((NS, COLS), jnp.float32),
            ]
            + [pltpu.SemaphoreType.DMA for _ in range(NBUF + 1)]
        ),
    )
    def _sc_segmean(inp_hbm, out_hbm, *scratch):
        bufs = scratch[:NBUF]
        acc, pbuf, shared = scratch[NBUF:NBUF + 3]
        sems = scratch[NBUF + 3:NBUF + 3 + NBUF]
        semo = scratch[NBUF + 3 + NBUF]
        c = lax.axis_index("c")
        s = lax.axis_index("s")
        seg_local = c * segs_per_core + s // w_per_seg
        sub = s % w_per_seg
        row0 = (seg0 + seg_local) * ROWS_PER_SEG + sub * rows_per_w

        def start(k, b):
            pltpu.make_async_copy(
                inp_hbm.at[pl.ds(row0 + k * RC, RC), :],
                bufs[b],
                sems[b],
            ).start()

        def wait(b):
            pltpu.make_async_copy(
                inp_hbm.at[pl.ds(row0, RC), :],
                bufs[b],
                sems[b],
            ).wait()

        def accum(buf):
            @plsc.parallel_loop(0, NV, unroll=2)
            def _jbody(j):
                cc = j * 16
                v = acc[pl.ds(cc, 16)]
                parts = []
                for g in range(RC // 8):
                    t = buf[g * 8, pl.ds(cc, 16)]
                    for r in range(g * 8 + 1, g * 8 + 8):
                        t = t + buf[r, pl.ds(cc, 16)]
                    parts.append(t)
                while len(parts) > 1:
                    parts = [a + b for a, b in zip(parts[::2], parts[1::2])]
                acc[pl.ds(cc, 16)] = v + parts[0]

        for b in range(NBUF):
            start(b, b)
        zero = jnp.zeros((16,), jnp.float32)
        for j in range(NV):
            acc[pl.ds(j * 16, 16)] = zero

        def ring(i, carry):
            for b in range(NBUF):
                k = i * NBUF + b
                wait(b)
                accum(bufs[b])
                start(k + NBUF, b)
            return carry
        lax.fori_loop(0, (nch - NBUF) // NBUF, ring, 0)

        for b in range(NBUF):
            wait(b)
            accum(bufs[b])

        pltpu.sync_copy(acc, shared.at[s])
        plsc.subcore_barrier()

        @pl.when(sub == 0)
        def _combine():
            for t in range(1, w_per_seg):
                pltpu.sync_copy(shared.at[s + t], pbuf)
                @plsc.parallel_loop(0, NV, unroll=2)
                def _addp(j):
                    cc = j * 16
                    acc[pl.ds(cc, 16)] = acc[pl.ds(cc, 16)] + pbuf[pl.ds(cc, 16)]
            scale = jnp.float32(1.0 / ROWS_PER_SEG)
            @plsc.parallel_loop(0, NV, unroll=2)
            def _scale(j):
                cc = j * 16
                acc[pl.ds(cc, 16)] = acc[pl.ds(cc, 16)] * scale
            pltpu.make_async_copy(acc, out_hbm.at[seg_local], semo).start()
            pltpu.make_async_copy(acc, out_hbm.at[seg_local], semo).wait()

    return _sc_segmean


if K_SC:
    _sc_call = _make_sc_segmean(K_SC, M_TC)


def kernel(inp):
    outs = []
    if M_TC:
        outs.append(_tc_segmean(inp))
    if K_SC:
        outs.append(_sc_call(inp))
    if len(outs) == 1:
        return outs[0]
    return jnp.concatenate(outs, axis=0)


# speedup vs baseline: 1.4677x; 1.0037x over previous
import functools

import jax
import jax.numpy as jnp
from jax import lax
from jax.experimental import pallas as pl
from jax.experimental.pallas import tpu as pltpu
from jax.experimental.pallas import tpu_sc as plsc

NUM_SEGMENTS = 16
ROWS_PER_SEG = 2048
COLS = 1024

TC_R = 1152
SC_R = ROWS_PER_SEG - TC_R


def _tc_body(x_ref, o_ref):
    i = pl.program_id(0)
    o_ref[pl.ds(i, 1), :] = jnp.sum(x_ref[0], axis=0, keepdims=True)


def _tc_segsum(inp3d):
    return pl.pallas_call(
        _tc_body,
        grid=(NUM_SEGMENTS,),
        in_specs=[pl.BlockSpec((1, TC_R, COLS), lambda i: (i, 0, 0))],
        out_specs=pl.BlockSpec((NUM_SEGMENTS, COLS), lambda i: (0, 0)),
        out_shape=jax.ShapeDtypeStruct((NUM_SEGMENTS, COLS), jnp.float32),
    )(inp3d)


NC = 2
NS = 16
NW = NC * NS
NV = COLS // 16
RC = 16
NBUF = 4

W_PER_SEG = NW // NUM_SEGMENTS
ROWS_PER_W = SC_R // W_PER_SEG
NCHUNK = ROWS_PER_W // RC
SEGS_PER_CORE = NUM_SEGMENTS // NC

assert SC_R % W_PER_SEG == 0 and ROWS_PER_W % (RC * NBUF) == 0


@functools.partial(
    pl.kernel,
    out_type=jax.ShapeDtypeStruct((NUM_SEGMENTS, COLS), jnp.float32),
    mesh=plsc.VectorSubcoreMesh(core_axis_name="c", subcore_axis_name="s"),
    scratch_types=(
        [pltpu.VMEM((RC, COLS), jnp.float32) for _ in range(NBUF)]
        + [
            pltpu.VMEM((COLS,), jnp.float32),
            pltpu.VMEM((COLS,), jnp.float32),
            pltpu.VMEM_SHARED((NS, COLS), jnp.float32),
        ]
        + [pltpu.SemaphoreType.DMA for _ in range(NBUF + 1)]
    ),
)
def _sc_segsum(inp_hbm, out_hbm, *scratch):
    bufs = scratch[:NBUF]
    acc, pbuf, shared = scratch[NBUF:NBUF + 3]
    sems = scratch[NBUF + 3:NBUF + 3 + NBUF]
    semo = scratch[NBUF + 3 + NBUF]

    c = lax.axis_index("c")
    s = lax.axis_index("s")
    seg_local = c * SEGS_PER_CORE + s // W_PER_SEG
    sub = s % W_PER_SEG
    row0 = seg_local * ROWS_PER_SEG + TC_R + sub * ROWS_PER_W

    def start(k, b):
        pltpu.make_async_copy(
            inp_hbm.at[pl.ds(row0 + k * RC, RC), :],
            bufs[b],
            sems[b],
        ).start()

    def wait(b):
        pltpu.make_async_copy(
            inp_hbm.at[pl.ds(row0, RC), :],
            bufs[b],
            sems[b],
        ).wait()

    def accum(buf):
        @plsc.parallel_loop(0, NV, unroll=2)
        def _jbody(j):
            cc = j * 16
            v = acc[pl.ds(cc, 16)]
            parts = []
            for g in range(RC // 8):
                t = buf[g * 8, pl.ds(cc, 16)]
                for r in range(g * 8 + 1, g * 8 + 8):
                    t = t + buf[r, pl.ds(cc, 16)]
                parts.append(t)
            while len(parts) > 1:
                parts = [a + b for a, b in zip(parts[::2], parts[1::2])]
            acc[pl.ds(cc, 16)] = v + parts[0]

    for b in range(NBUF):
        start(b, b)
    zero = jnp.zeros((16,), jnp.float32)
    for j in range(NV):
        acc[pl.ds(j * 16, 16)] = zero

    def ring(i, carry):
        for b in range(NBUF):
            k = i * NBUF + b
            wait(b)
            accum(bufs[b])
            start(k + NBUF, b)
        return carry
    lax.fori_loop(0, (NCHUNK - NBUF) // NBUF, ring, 0)

    for b in range(NBUF):
        wait(b)
        accum(bufs[b])

    pltpu.sync_copy(acc, shared.at[s])
    plsc.subcore_barrier()

    @pl.when(sub == 0)
    def _combine():
        for t in range(1, W_PER_SEG):
            pltpu.sync_copy(shared.at[s + t], pbuf)

            @plsc.parallel_loop(0, NV, unroll=2)
            def _addp(j):
                cc = j * 16
                acc[pl.ds(cc, 16)] = acc[pl.ds(cc, 16)] + pbuf[pl.ds(cc, 16)]

        pltpu.make_async_copy(acc, out_hbm.at[seg_local], semo).start()
        pltpu.make_async_copy(acc, out_hbm.at[seg_local], semo).wait()


def _merge_body(a_ref, b_ref, o_ref):
    o_ref[...] = (a_ref[...] + b_ref[...]) * jnp.float32(1.0 / ROWS_PER_SEG)


def _merge(a, b):
    return pl.pallas_call(
        _merge_body,
        out_shape=jax.ShapeDtypeStruct((NUM_SEGMENTS, COLS), jnp.float32),
    )(a, b)


def kernel(inp):
    tc_sums = _tc_segsum(inp.reshape(NUM_SEGMENTS, ROWS_PER_SEG, COLS))
    sc_sums = _sc_segsum(inp)
    return _merge(tc_sums, sc_sums)


# speedup vs baseline: 1.4684x; 1.0005x over previous
import functools

import jax
import jax.numpy as jnp
from jax import lax
from jax.experimental import pallas as pl
from jax.experimental.pallas import tpu as pltpu
from jax.experimental.pallas import tpu_sc as plsc

NUM_SEGMENTS = 16
ROWS_PER_SEG = 2048
COLS = 1024

TC_R = 1152
SC_R = ROWS_PER_SEG - TC_R


def _tc_body(x_ref, o_ref):
    i = pl.program_id(0)
    o_ref[pl.ds(i, 1), :] = jnp.sum(x_ref[0], axis=0, keepdims=True)


def _tc_segsum(inp3d):
    return pl.pallas_call(
        _tc_body,
        grid=(NUM_SEGMENTS,),
        in_specs=[pl.BlockSpec((1, TC_R, COLS), lambda i: (i, 0, 0))],
        out_specs=pl.BlockSpec((NUM_SEGMENTS, COLS), lambda i: (0, 0)),
        out_shape=jax.ShapeDtypeStruct((NUM_SEGMENTS, COLS), jnp.float32),
    )(inp3d)


NC = 2
NS = 16
NW = NC * NS
NV = COLS // 16
RC = 32
NBUF = 2

W_PER_SEG = NW // NUM_SEGMENTS
ROWS_PER_W = SC_R // W_PER_SEG
NCHUNK = ROWS_PER_W // RC
SEGS_PER_CORE = NUM_SEGMENTS // NC

assert SC_R % W_PER_SEG == 0 and ROWS_PER_W % (RC * NBUF) == 0


@functools.partial(
    pl.kernel,
    out_type=jax.ShapeDtypeStruct((NUM_SEGMENTS, COLS), jnp.float32),
    mesh=plsc.VectorSubcoreMesh(core_axis_name="c", subcore_axis_name="s"),
    scratch_types=(
        [pltpu.VMEM((RC, COLS), jnp.float32) for _ in range(NBUF)]
        + [
            pltpu.VMEM((COLS,), jnp.float32),
            pltpu.VMEM((COLS,), jnp.float32),
            pltpu.VMEM_SHARED((NS, COLS), jnp.float32),
        ]
        + [pltpu.SemaphoreType.DMA for _ in range(NBUF + 1)]
    ),
)
def _sc_segsum(inp_hbm, out_hbm, *scratch):
    bufs = scratch[:NBUF]
    acc, pbuf, shared = scratch[NBUF:NBUF + 3]
    sems = scratch[NBUF + 3:NBUF + 3 + NBUF]
    semo = scratch[NBUF + 3 + NBUF]

    c = lax.axis_index("c")
    s = lax.axis_index("s")
    seg_local = c * SEGS_PER_CORE + s // W_PER_SEG
    sub = s % W_PER_SEG
    row0 = seg_local * ROWS_PER_SEG + TC_R + sub * ROWS_PER_W

    def start(k, b):
        pltpu.make_async_copy(
            inp_hbm.at[pl.ds(row0 + k * RC, RC), :],
            bufs[b],
            sems[b],
        ).start()

    def wait(b):
        pltpu.make_async_copy(
            inp_hbm.at[pl.ds(row0, RC), :],
            bufs[b],
            sems[b],
        ).wait()

    def accum(buf):
        @plsc.parallel_loop(0, NV, unroll=2)
        def _jbody(j):
            cc = j * 16
            v = acc[pl.ds(cc, 16)]
            parts = []
            for g in range(RC // 8):
                t = buf[g * 8, pl.ds(cc, 16)]
                for r in range(g * 8 + 1, g * 8 + 8):
                    t = t + buf[r, pl.ds(cc, 16)]
                parts.append(t)
            while len(parts) > 1:
                parts = [a + b for a, b in zip(parts[::2], parts[1::2])]
            acc[pl.ds(cc, 16)] = v + parts[0]

    for b in range(NBUF):
        start(b, b)
    zero = jnp.zeros((16,), jnp.float32)
    for j in range(NV):
        acc[pl.ds(j * 16, 16)] = zero

    def ring(i, carry):
        for b in range(NBUF):
            k = i * NBUF + b
            wait(b)
            accum(bufs[b])
            start(k + NBUF, b)
        return carry
    lax.fori_loop(0, (NCHUNK - NBUF) // NBUF, ring, 0)

    for b in range(NBUF):
        wait(b)
        accum(bufs[b])

    pltpu.sync_copy(acc, shared.at[s])
    plsc.subcore_barrier()

    @pl.when(sub == 0)
    def _combine():
        for t in range(1, W_PER_SEG):
            pltpu.sync_copy(shared.at[s + t], pbuf)

            @plsc.parallel_loop(0, NV, unroll=2)
            def _addp(j):
                cc = j * 16
                acc[pl.ds(cc, 16)] = acc[pl.ds(cc, 16)] + pbuf[pl.ds(cc, 16)]

        pltpu.make_async_copy(acc, out_hbm.at[seg_local], semo).start()
        pltpu.make_async_copy(acc, out_hbm.at[seg_local], semo).wait()


def _merge_body(a_ref, b_ref, o_ref):
    o_ref[...] = (a_ref[...] + b_ref[...]) * jnp.float32(1.0 / ROWS_PER_SEG)


def _merge(a, b):
    return pl.pallas_call(
        _merge_body,
        out_shape=jax.ShapeDtypeStruct((NUM_SEGMENTS, COLS), jnp.float32),
    )(a, b)


def kernel(inp):
    tc_sums = _tc_segsum(inp.reshape(NUM_SEGMENTS, ROWS_PER_SEG, COLS))
    sc_sums = _sc_segsum(inp)
    return _merge(tc_sums, sc_sums)


# speedup vs baseline: 1.4717x; 1.0022x over previous
import functools

import jax
import jax.numpy as jnp
from jax import lax
from jax.experimental import pallas as pl
from jax.experimental.pallas import tpu as pltpu
from jax.experimental.pallas import tpu_sc as plsc

NUM_SEGMENTS = 16
ROWS_PER_SEG = 2048
COLS = 1024

TC_R = 1152
SC_R = ROWS_PER_SEG - TC_R


TC_SPB = 2


def _tc_body(x_ref, o_ref):
    i = pl.program_id(0)
    sums = jnp.sum(x_ref[...], axis=1)
    for t in range(TC_SPB):
        o_ref[pl.ds(i * TC_SPB + t, 1), :] = sums[t:t + 1]


def _tc_segsum(inp3d):
    return pl.pallas_call(
        _tc_body,
        grid=(NUM_SEGMENTS // TC_SPB,),
        in_specs=[pl.BlockSpec((TC_SPB, TC_R, COLS), lambda i: (i, 0, 0))],
        out_specs=pl.BlockSpec((NUM_SEGMENTS, COLS), lambda i: (0, 0)),
        out_shape=jax.ShapeDtypeStruct((NUM_SEGMENTS, COLS), jnp.float32),
    )(inp3d)


NC = 2
NS = 16
NW = NC * NS
NV = COLS // 16
RC = 32
NBUF = 2

W_PER_SEG = NW // NUM_SEGMENTS
ROWS_PER_W = SC_R // W_PER_SEG
NCHUNK = ROWS_PER_W // RC
SEGS_PER_CORE = NUM_SEGMENTS // NC

assert SC_R % W_PER_SEG == 0 and ROWS_PER_W % (RC * NBUF) == 0


@functools.partial(
    pl.kernel,
    out_type=jax.ShapeDtypeStruct((NUM_SEGMENTS, COLS), jnp.float32),
    mesh=plsc.VectorSubcoreMesh(core_axis_name="c", subcore_axis_name="s"),
    scratch_types=(
        [pltpu.VMEM((RC, COLS), jnp.float32) for _ in range(NBUF)]
        + [
            pltpu.VMEM((COLS,), jnp.float32),
            pltpu.VMEM((COLS,), jnp.float32),
            pltpu.VMEM_SHARED((NS, COLS), jnp.float32),
        ]
        + [pltpu.SemaphoreType.DMA for _ in range(NBUF + 1)]
    ),
)
def _sc_segsum(inp_hbm, out_hbm, *scratch):
    bufs = scratch[:NBUF]
    acc, pbuf, shared = scratch[NBUF:NBUF + 3]
    sems = scratch[NBUF + 3:NBUF + 3 + NBUF]
    semo = scratch[NBUF + 3 + NBUF]

    c = lax.axis_index("c")
    s = lax.axis_index("s")
    seg_local = c * SEGS_PER_CORE + s // W_PER_SEG
    sub = s % W_PER_SEG
    row0 = seg_local * ROWS_PER_SEG + TC_R + sub * ROWS_PER_W

    def start(k, b):
        pltpu.make_async_copy(
            inp_hbm.at[pl.ds(row0 + k * RC, RC), :],
            bufs[b],
            sems[b],
        ).start()

    def wait(b):
        pltpu.make_async_copy(
            inp_hbm.at[pl.ds(row0, RC), :],
            bufs[b],
            sems[b],
        ).wait()

    def accum(buf):
        @plsc.parallel_loop(0, NV, unroll=2)
        def _jbody(j):
            cc = j * 16
            v = acc[pl.ds(cc, 16)]
            parts = []
            for g in range(RC // 8):
                t = buf[g * 8, pl.ds(cc, 16)]
                for r in range(g * 8 + 1, g * 8 + 8):
                    t = t + buf[r, pl.ds(cc, 16)]
                parts.append(t)
            while len(parts) > 1:
                parts = [a + b for a, b in zip(parts[::2], parts[1::2])]
            acc[pl.ds(cc, 16)] = v + parts[0]

    for b in range(NBUF):
        start(b, b)
    zero = jnp.zeros((16,), jnp.float32)
    for j in range(NV):
        acc[pl.ds(j * 16, 16)] = zero

    def ring(i, carry):
        for b in range(NBUF):
            k = i * NBUF + b
            wait(b)
            accum(bufs[b])
            start(k + NBUF, b)
        return carry
    lax.fori_loop(0, (NCHUNK - NBUF) // NBUF, ring, 0)

    for b in range(NBUF):
        wait(b)
        accum(bufs[b])

    pltpu.sync_copy(acc, shared.at[s])
    plsc.subcore_barrier()

    @pl.when(sub == 0)
    def _combine():
        for t in range(1, W_PER_SEG):
            pltpu.sync_copy(shared.at[s + t], pbuf)

            @plsc.parallel_loop(0, NV, unroll=2)
            def _addp(j):
                cc = j * 16
                acc[pl.ds(cc, 16)] = acc[pl.ds(cc, 16)] + pbuf[pl.ds(cc, 16)]

        pltpu.make_async_copy(acc, out_hbm.at[seg_local], semo).start()
        pltpu.make_async_copy(acc, out_hbm.at[seg_local], semo).wait()


def _merge_body(a_ref, b_ref, o_ref):
    o_ref[...] = (a_ref[...] + b_ref[...]) * jnp.float32(1.0 / ROWS_PER_SEG)


def _merge(a, b):
    return pl.pallas_call(
        _merge_body,
        out_shape=jax.ShapeDtypeStruct((NUM_SEGMENTS, COLS), jnp.float32),
    )(a, b)


def kernel(inp):
    tc_sums = _tc_segsum(inp.reshape(NUM_SEGMENTS, ROWS_PER_SEG, COLS))
    sc_sums = _sc_segsum(inp)
    return _merge(tc_sums, sc_sums)


# speedup vs baseline: 1.5067x; 1.0238x over previous
import functools

import jax
import jax.numpy as jnp
from jax import lax
from jax.experimental import pallas as pl
from jax.experimental.pallas import tpu as pltpu
from jax.experimental.pallas import tpu_sc as plsc

NUM_SEGMENTS = 16
ROWS_PER_SEG = 2048
COLS = 1024

TC_R = 1216
SC_R = ROWS_PER_SEG - TC_R


TC_SPB = 2


def _tc_body(x_ref, o_ref):
    i = pl.program_id(0)
    sums = jnp.sum(x_ref[...], axis=1)
    for t in range(TC_SPB):
        o_ref[pl.ds(i * TC_SPB + t, 1), :] = sums[t:t + 1]


def _tc_segsum(inp3d):
    return pl.pallas_call(
        _tc_body,
        grid=(NUM_SEGMENTS // TC_SPB,),
        in_specs=[pl.BlockSpec((TC_SPB, TC_R, COLS), lambda i: (i, 0, 0))],
        out_specs=pl.BlockSpec((NUM_SEGMENTS, COLS), lambda i: (0, 0)),
        out_shape=jax.ShapeDtypeStruct((NUM_SEGMENTS, COLS), jnp.float32),
    )(inp3d)


NC = 2
NS = 16
NW = NC * NS
NV = COLS // 16
RC = 16
NBUF = 2

W_PER_SEG = NW // NUM_SEGMENTS
ROWS_PER_W = SC_R // W_PER_SEG
NCHUNK = ROWS_PER_W // RC
SEGS_PER_CORE = NUM_SEGMENTS // NC

assert SC_R % W_PER_SEG == 0 and ROWS_PER_W % (RC * NBUF) == 0


@functools.partial(
    pl.kernel,
    out_type=jax.ShapeDtypeStruct((NUM_SEGMENTS, COLS), jnp.float32),
    mesh=plsc.VectorSubcoreMesh(core_axis_name="c", subcore_axis_name="s"),
    scratch_types=(
        [pltpu.VMEM((RC, COLS), jnp.float32) for _ in range(NBUF)]
        + [
            pltpu.VMEM((COLS,), jnp.float32),
            pltpu.VMEM((COLS,), jnp.float32),
            pltpu.VMEM_SHARED((NS, COLS), jnp.float32),
        ]
        + [pltpu.SemaphoreType.DMA for _ in range(NBUF + 1)]
    ),
)
def _sc_segsum(inp_hbm, out_hbm, *scratch):
    bufs = scratch[:NBUF]
    acc, pbuf, shared = scratch[NBUF:NBUF + 3]
    sems = scratch[NBUF + 3:NBUF + 3 + NBUF]
    semo = scratch[NBUF + 3 + NBUF]

    c = lax.axis_index("c")
    s = lax.axis_index("s")
    seg_local = c * SEGS_PER_CORE + s // W_PER_SEG
    sub = s % W_PER_SEG
    row0 = seg_local * ROWS_PER_SEG + TC_R + sub * ROWS_PER_W

    def start(k, b):
        pltpu.make_async_copy(
            inp_hbm.at[pl.ds(row0 + k * RC, RC), :],
            bufs[b],
            sems[b],
        ).start()

    def wait(b):
        pltpu.make_async_copy(
            inp_hbm.at[pl.ds(row0, RC), :],
            bufs[b],
            sems[b],
        ).wait()

    def accum(buf):
        @plsc.parallel_loop(0, NV, unroll=2)
        def _jbody(j):
            cc = j * 16
            v = acc[pl.ds(cc, 16)]
            parts = []
            for g in range(RC // 8):
                t = buf[g * 8, pl.ds(cc, 16)]
                for r in range(g * 8 + 1, g * 8 + 8):
                    t = t + buf[r, pl.ds(cc, 16)]
                parts.append(t)
            while len(parts) > 1:
                parts = [a + b for a, b in zip(parts[::2], parts[1::2])]
            acc[pl.ds(cc, 16)] = v + parts[0]

    for b in range(NBUF):
        start(b, b)
    zero = jnp.zeros((16,), jnp.float32)
    for j in range(NV):
        acc[pl.ds(j * 16, 16)] = zero

    def ring(i, carry):
        for b in range(NBUF):
            k = i * NBUF + b
            wait(b)
            accum(bufs[b])
            start(k + NBUF, b)
        return carry
    lax.fori_loop(0, (NCHUNK - NBUF) // NBUF, ring, 0)

    for b in range(NBUF):
        wait(b)
        accum(bufs[b])

    pltpu.sync_copy(acc, shared.at[s])
    plsc.subcore_barrier()

    @pl.when(sub == 0)
    def _combine():
        for t in range(1, W_PER_SEG):
            pltpu.sync_copy(shared.at[s + t], pbuf)

            @plsc.parallel_loop(0, NV, unroll=2)
            def _addp(j):
                cc = j * 16
                acc[pl.ds(cc, 16)] = acc[pl.ds(cc, 16)] + pbuf[pl.ds(cc, 16)]

        pltpu.make_async_copy(acc, out_hbm.at[seg_local], semo).start()
        pltpu.make_async_copy(acc, out_hbm.at[seg_local], semo).wait()


def _merge_body(a_ref, b_ref, o_ref):
    o_ref[...] = (a_ref[...] + b_ref[...]) * jnp.float32(1.0 / ROWS_PER_SEG)


def _merge(a, b):
    return pl.pallas_call(
        _merge_body,
        out_shape=jax.ShapeDtypeStruct((NUM_SEGMENTS, COLS), jnp.float32),
    )(a, b)


def kernel(inp):
    tc_sums = _tc_segsum(inp.reshape(NUM_SEGMENTS, ROWS_PER_SEG, COLS))
    sc_sums = _sc_segsum(inp)
    return _merge(tc_sums, sc_sums)


# speedup vs baseline: 1.5413x; 1.0229x over previous
import functools

import jax
import jax.numpy as jnp
from jax import lax
from jax.experimental import pallas as pl
from jax.experimental.pallas import tpu as pltpu
from jax.experimental.pallas import tpu_sc as plsc

NUM_SEGMENTS = 16
ROWS_PER_SEG = 2048
COLS = 1024

TC_R = 1280
SC_R = ROWS_PER_SEG - TC_R


TC_SPB = 2


def _tc_body(x_ref, o_ref):
    i = pl.program_id(0)
    sums = jnp.sum(x_ref[...], axis=1)
    for t in range(TC_SPB):
        o_ref[pl.ds(i * TC_SPB + t, 1), :] = sums[t:t + 1]


def _tc_segsum(inp3d):
    return pl.pallas_call(
        _tc_body,
        grid=(NUM_SEGMENTS // TC_SPB,),
        in_specs=[pl.BlockSpec((TC_SPB, TC_R, COLS), lambda i: (i, 0, 0))],
        out_specs=pl.BlockSpec((NUM_SEGMENTS, COLS), lambda i: (0, 0)),
        out_shape=jax.ShapeDtypeStruct((NUM_SEGMENTS, COLS), jnp.float32),
    )(inp3d)


NC = 2
NS = 16
NW = NC * NS
NV = COLS // 16
RC = 16
NBUF = 2

W_PER_SEG = NW // NUM_SEGMENTS
ROWS_PER_W = SC_R // W_PER_SEG
NCHUNK = ROWS_PER_W // RC
SEGS_PER_CORE = NUM_SEGMENTS // NC

assert SC_R % W_PER_SEG == 0 and ROWS_PER_W % (RC * NBUF) == 0


@functools.partial(
    pl.kernel,
    out_type=jax.ShapeDtypeStruct((NUM_SEGMENTS, COLS), jnp.float32),
    mesh=plsc.VectorSubcoreMesh(core_axis_name="c", subcore_axis_name="s"),
    scratch_types=(
        [pltpu.VMEM((RC, COLS), jnp.float32) for _ in range(NBUF)]
        + [
            pltpu.VMEM((COLS,), jnp.float32),
            pltpu.VMEM((COLS,), jnp.float32),
            pltpu.VMEM_SHARED((NS, COLS), jnp.float32),
        ]
        + [pltpu.SemaphoreType.DMA for _ in range(NBUF + 1)]
    ),
)
def _sc_segsum(inp_hbm, out_hbm, *scratch):
    bufs = scratch[:NBUF]
    acc, pbuf, shared = scratch[NBUF:NBUF + 3]
    sems = scratch[NBUF + 3:NBUF + 3 + NBUF]
    semo = scratch[NBUF + 3 + NBUF]

    c = lax.axis_index("c")
    s = lax.axis_index("s")
    seg_local = c * SEGS_PER_CORE + s // W_PER_SEG
    sub = s % W_PER_SEG
    row0 = seg_local * ROWS_PER_SEG + TC_R + sub * ROWS_PER_W

    def start(k, b):
        pltpu.make_async_copy(
            inp_hbm.at[pl.ds(row0 + k * RC, RC), :],
            bufs[b],
            sems[b],
        ).start()

    def wait(b):
        pltpu.make_async_copy(
            inp_hbm.at[pl.ds(row0, RC), :],
            bufs[b],
            sems[b],
        ).wait()

    def accum(buf):
        @plsc.parallel_loop(0, NV, unroll=2)
        def _jbody(j):
            cc = j * 16
            v = acc[pl.ds(cc, 16)]
            parts = []
            for g in range(RC // 8):
                t = buf[g * 8, pl.ds(cc, 16)]
                for r in range(g * 8 + 1, g * 8 + 8):
                    t = t + buf[r, pl.ds(cc, 16)]
                parts.append(t)
            while len(parts) > 1:
                parts = [a + b for a, b in zip(parts[::2], parts[1::2])]
            acc[pl.ds(cc, 16)] = v + parts[0]

    for b in range(NBUF):
        start(b, b)
    zero = jnp.zeros((16,), jnp.float32)
    for j in range(NV):
        acc[pl.ds(j * 16, 16)] = zero

    def ring(i, carry):
        for b in range(NBUF):
            k = i * NBUF + b
            wait(b)
            accum(bufs[b])
            start(k + NBUF, b)
        return carry
    lax.fori_loop(0, (NCHUNK - NBUF) // NBUF, ring, 0)

    for b in range(NBUF):
        wait(b)
        accum(bufs[b])

    pltpu.sync_copy(acc, shared.at[s])
    plsc.subcore_barrier()

    @pl.when(sub == 0)
    def _combine():
        for t in range(1, W_PER_SEG):
            pltpu.sync_copy(shared.at[s + t], pbuf)

            @plsc.parallel_loop(0, NV, unroll=2)
            def _addp(j):
                cc = j * 16
                acc[pl.ds(cc, 16)] = acc[pl.ds(cc, 16)] + pbuf[pl.ds(cc, 16)]

        pltpu.make_async_copy(acc, out_hbm.at[seg_local], semo).start()
        pltpu.make_async_copy(acc, out_hbm.at[seg_local], semo).wait()


def _merge_body(a_ref, b_ref, o_ref):
    o_ref[...] = (a_ref[...] + b_ref[...]) * jnp.float32(1.0 / ROWS_PER_SEG)


def _merge(a, b):
    return pl.pallas_call(
        _merge_body,
        out_shape=jax.ShapeDtypeStruct((NUM_SEGMENTS, COLS), jnp.float32),
    )(a, b)


def kernel(inp):
    tc_sums = _tc_segsum(inp.reshape(NUM_SEGMENTS, ROWS_PER_SEG, COLS))
    sc_sums = _sc_segsum(inp)
    return _merge(tc_sums, sc_sums)
